# Initial kernel scaffold; baseline (speedup 1.0000x reference)
import jax
import jax.numpy as jnp
from jax.experimental import pallas as pl


def kernel(x, edge_index, W_pre, b_pre, ln_g, ln_b, Wl, bl, Wr, ln_gf, ln_bf, W_post, b_post):
    raise NotImplementedError("write your pallas kernel here")


# speedup vs baseline: 2.7888x; 2.7888x over previous
import functools

import jax
import jax.numpy as jnp
from jax import lax
from jax.experimental import pallas as pl
from jax.experimental.pallas import tpu as pltpu
from jax.experimental.pallas import tpu_sc as plsc

N = 10000
E = 320000
D = 128
L = 3

NC = 2
NS = 16
NW = NC * NS
CK = 128
NCHUNK = 80
EPW = NCHUNK * CK
EPAD = NW * EPW
NPAD = 10112
DUMMY = N

def _sc_aggregate_body(u_hbm, src_hbm, dst_hbm, zeros_hbm, out_hbm,
                       src_v, dst_v, rows_v, acc_sh, sem):
    c = lax.axis_index("c")
    s = lax.axis_index("s")
    wid = s * NC + c
    zrows = NPAD // NS
    pltpu.sync_copy(zeros_hbm.at[pl.ds(s * zrows, zrows)],
                    acc_sh.at[pl.ds(s * zrows, zrows)])
    pltpu.sync_copy(src_hbm.at[wid], src_v)
    pltpu.sync_copy(dst_hbm.at[wid], dst_v)
    plsc.subcore_barrier()

    def body(j, carry):
        pltpu.async_copy(u_hbm.at[src_v.at[j]], rows_v, sem).wait()
        pltpu.sync_copy(rows_v, acc_sh.at[dst_v.at[j]], add=True)
        return carry

    lax.fori_loop(0, NCHUNK, body, 0)
    plsc.subcore_barrier()
    orows = NPAD // NS
    pltpu.sync_copy(acc_sh.at[pl.ds(s * orows, orows)],
                    out_hbm.at[c].at[pl.ds(s * orows, orows)])


def _sc_degree_body(dst_hbm, zeros_hbm, ones_hbm, out_hbm,
                    dst_v, ones_v, acc_sh):
    c = lax.axis_index("c")
    s = lax.axis_index("s")
    wid = s * NC + c
    zrows = NPAD // NS
    pltpu.sync_copy(zeros_hbm.at[pl.ds(s * zrows, zrows)],
                    acc_sh.at[pl.ds(s * zrows, zrows)])
    pltpu.sync_copy(dst_hbm.at[wid], dst_v)
    pltpu.sync_copy(ones_hbm, ones_v)
    plsc.subcore_barrier()

    def body(j, carry):
        pltpu.sync_copy(ones_v, acc_sh.at[dst_v.at[j]], add=True)
        return carry

    lax.fori_loop(0, NCHUNK, body, 0)
    plsc.subcore_barrier()
    orows = NPAD // NS
    pltpu.sync_copy(acc_sh.at[pl.ds(s * orows, orows)],
                    out_hbm.at[c].at[pl.ds(s * orows, orows)])


@functools.cache
def _sc_kernels():
    mesh = plsc.VectorSubcoreMesh(core_axis_name="c", subcore_axis_name="s")
    sc_aggregate = functools.partial(
        pl.kernel,
        mesh=mesh,
        out_type=jax.ShapeDtypeStruct((NC, NPAD, D), jnp.float32),
        scratch_types=[
            pltpu.VMEM((NCHUNK, CK), jnp.int32),
            pltpu.VMEM((NCHUNK, CK), jnp.int32),
            pltpu.VMEM((CK, D), jnp.float32),
            pltpu.VMEM_SHARED((NPAD, D), jnp.float32),
            pltpu.SemaphoreType.DMA,
        ],
    )(_sc_aggregate_body)
    sc_degree = functools.partial(
        pl.kernel,
        mesh=mesh,
        out_type=jax.ShapeDtypeStruct((NC, NPAD, D), jnp.float32),
        scratch_types=[
            pltpu.VMEM((NCHUNK, CK), jnp.int32),
            pltpu.VMEM((CK, D), jnp.float32),
            pltpu.VMEM_SHARED((NPAD, D), jnp.float32),
        ],
    )(_sc_degree_body)
    return sc_aggregate, sc_degree


ROWS = 400
GRID = N // ROWS
_HI = lax.Precision.HIGHEST


def _ln_relu(h, g, b):
    m = jnp.mean(h, axis=-1, keepdims=True)
    d = h - m
    v = jnp.mean(d * d, axis=-1, keepdims=True)
    y = d * lax.rsqrt(v + 1e-5) * g + b
    return jnp.maximum(y, 0.0)


def _tc_pre_body(x_ref, wpre_ref, bpre_ref, g_ref, b_ref, h_ref, u_ref):
    h = jnp.dot(x_ref[...], wpre_ref[...], precision=_HI) + bpre_ref[...]
    h_ref[...] = h
    u_ref[...] = _ln_relu(h, g_ref[...], b_ref[...])


def _agg_update(s0, s1, d0, d1, u, h, wl, bl, wr):
    cnt = d0[:, 0:1] + d1[:, 0:1]
    inv = 1.0 / jnp.maximum(cnt, 1.0)
    agg = (s0 + s1) * inv
    return (jnp.dot(agg, wl, precision=_HI) + bl
            + jnp.dot(u, wr, precision=_HI) + h)


def _tc_mid_body(s0_ref, s1_ref, d0_ref, d1_ref, u_ref, h_ref,
                 wl_ref, bl_ref, wr_ref, g_ref, b_ref, ho_ref, uo_ref):
    hn = _agg_update(s0_ref[...], s1_ref[...], d0_ref[...], d1_ref[...],
                     u_ref[...], h_ref[...], wl_ref[...], bl_ref[...],
                     wr_ref[...])
    ho_ref[...] = hn
    uo_ref[...] = _ln_relu(hn, g_ref[...], b_ref[...])


def _tc_final_body(s0_ref, s1_ref, d0_ref, d1_ref, u_ref, h_ref,
                   wl_ref, bl_ref, wr_ref, gf_ref, bf_ref,
                   wpost_ref, bpost_ref, out_ref):
    hn = _agg_update(s0_ref[...], s1_ref[...], d0_ref[...], d1_ref[...],
                     u_ref[...], h_ref[...], wl_ref[...], bl_ref[...],
                     wr_ref[...])
    t = _ln_relu(hn, gf_ref[...], bf_ref[...])
    out_ref[...] = jnp.dot(t, wpost_ref[...], precision=_HI) + bpost_ref[...]


def _row_spec():
    return pl.BlockSpec((ROWS, D), lambda i: (i, 0))


def _deg_spec():
    return pl.BlockSpec((ROWS, 16), lambda i: (i, 0))


def _w_spec():
    return pl.BlockSpec((D, D), lambda i: (0, 0))


def _b_spec():
    return pl.BlockSpec((1, D), lambda i: (0, 0))


_tc_pre = pl.pallas_call(
    _tc_pre_body,
    grid=(GRID,),
    in_specs=[_row_spec(), _w_spec(), _b_spec(), _b_spec(), _b_spec()],
    out_specs=[_row_spec(), _row_spec()],
    out_shape=[jax.ShapeDtypeStruct((N, D), jnp.float32),
               jax.ShapeDtypeStruct((N, D), jnp.float32)],
)

_tc_mid = pl.pallas_call(
    _tc_mid_body,
    grid=(GRID,),
    in_specs=[_row_spec(), _row_spec(), _deg_spec(), _deg_spec(),
              _row_spec(), _row_spec(), _w_spec(), _b_spec(), _w_spec(),
              _b_spec(), _b_spec()],
    out_specs=[_row_spec(), _row_spec()],
    out_shape=[jax.ShapeDtypeStruct((N, D), jnp.float32),
               jax.ShapeDtypeStruct((N, D), jnp.float32)],
)

_tc_final = pl.pallas_call(
    _tc_final_body,
    grid=(GRID,),
    in_specs=[_row_spec(), _row_spec(), _deg_spec(), _deg_spec(),
              _row_spec(), _row_spec(), _w_spec(), _b_spec(), _w_spec(),
              _b_spec(), _b_spec(), _w_spec(), _b_spec()],
    out_specs=_row_spec(),
    out_shape=jax.ShapeDtypeStruct((N, D), jnp.float32),
)


def kernel(x, edge_index, W_pre, b_pre, ln_g, ln_b, Wl, bl, Wr,
           ln_gf, ln_bf, W_post, b_post):
    src = edge_index[0]
    dst = edge_index[1]
    pad = EPAD - E
    src_p = jnp.concatenate(
        [src, jnp.zeros((pad,), jnp.int32)]).reshape(NW, NCHUNK, CK)
    dst_p = jnp.concatenate(
        [dst, jnp.full((pad,), DUMMY, jnp.int32)]).reshape(NW, NCHUNK, CK)
    zeros_big = jnp.zeros((NPAD, D), jnp.float32)
    ones_deg = jnp.ones((CK, D), jnp.float32)

    sc_aggregate, sc_degree = _sc_kernels()
    deg = sc_degree(dst_p, zeros_big, ones_deg)
    d0 = deg[0, :, :16]
    d1 = deg[1, :, :16]

    h, u = _tc_pre(x, W_pre.T, b_pre[None], ln_g[0][None], ln_b[0][None])
    out = None
    for l in range(L):
        S = sc_aggregate(u, src_p, dst_p, zeros_big)
        if l < L - 1:
            h, u = _tc_mid(S[0], S[1], d0, d1, u, h,
                           Wl[l].T, bl[l][None], Wr[l].T,
                           ln_g[l + 1][None], ln_b[l + 1][None])
        else:
            out = _tc_final(S[0], S[1], d0, d1, u, h,
                            Wl[l].T, bl[l][None], Wr[l].T,
                            ln_gf[None], ln_bf[None],
                            W_post.T, b_post[None])
    return out


# speedup vs baseline: 3.1022x; 1.1124x over previous
import functools

import jax
import jax.numpy as jnp
from jax import lax
from jax.experimental import pallas as pl
from jax.experimental.pallas import tpu as pltpu
from jax.experimental.pallas import tpu_sc as plsc

N = 10000
E = 320000
D = 128
L = 3

NC = 2
NS = 16
NW = NC * NS
CK = 128
NCHUNK = 80
EPW = NCHUNK * CK
EPAD = NW * EPW
NPAD = 10112
DUMMY = N
NSTAGE = 2
CPS = NCHUNK // NSTAGE

def _sc_aggregate_body(u_hbm, src_hbm, dst_hbm, zeros_hbm, out_hbm,
                       src_v, dst_v, rows_a, rows_b, acc_sh, sema, semb):
    c = lax.axis_index("c")
    s = lax.axis_index("s")
    wid = s * NC + c
    zrows = NPAD // NS
    pltpu.sync_copy(zeros_hbm.at[pl.ds(s * zrows, zrows)],
                    acc_sh.at[pl.ds(s * zrows, zrows)])
    plsc.subcore_barrier()

    for st in range(NSTAGE):
        pltpu.sync_copy(src_hbm.at[wid].at[pl.ds(st * CPS, CPS)], src_v)
        pltpu.sync_copy(dst_hbm.at[wid].at[pl.ds(st * CPS, CPS)], dst_v)
        pltpu.async_copy(u_hbm.at[src_v.at[0]], rows_a, sema)

        def body(i, carry):
            j0 = 2 * i
            pltpu.async_copy(u_hbm.at[src_v.at[j0 + 1]], rows_b, semb)
            pltpu.make_async_copy(u_hbm.at[src_v.at[j0]], rows_a, sema).wait()
            pltpu.sync_copy(rows_a, acc_sh.at[dst_v.at[j0]], add=True)
            pltpu.async_copy(u_hbm.at[src_v.at[j0 + 2]], rows_a, sema)
            pltpu.make_async_copy(u_hbm.at[src_v.at[j0 + 1]], rows_b,
                                  semb).wait()
            pltpu.sync_copy(rows_b, acc_sh.at[dst_v.at[j0 + 1]], add=True)
            return carry

        lax.fori_loop(0, (CPS - 2) // 2, body, 0)
        jlast = CPS - 2
        pltpu.async_copy(u_hbm.at[src_v.at[jlast + 1]], rows_b, semb)
        pltpu.make_async_copy(u_hbm.at[src_v.at[jlast]], rows_a, sema).wait()
        pltpu.sync_copy(rows_a, acc_sh.at[dst_v.at[jlast]], add=True)
        pltpu.make_async_copy(u_hbm.at[src_v.at[jlast + 1]], rows_b,
                              semb).wait()
        pltpu.sync_copy(rows_b, acc_sh.at[dst_v.at[jlast + 1]], add=True)
    plsc.subcore_barrier()
    orows = NPAD // NS
    pltpu.sync_copy(acc_sh.at[pl.ds(s * orows, orows)],
                    out_hbm.at[c].at[pl.ds(s * orows, orows)])


def _sc_degree_body(dst_hbm, zeros_hbm, ones_hbm, out_hbm,
                    dst_v, ones_v, acc_sh):
    c = lax.axis_index("c")
    s = lax.axis_index("s")
    wid = s * NC + c
    zrows = NPAD // NS
    pltpu.sync_copy(zeros_hbm.at[pl.ds(s * zrows, zrows)],
                    acc_sh.at[pl.ds(s * zrows, zrows)])
    pltpu.sync_copy(dst_hbm.at[wid], dst_v)
    pltpu.sync_copy(ones_hbm, ones_v)
    plsc.subcore_barrier()

    def body(j, carry):
        pltpu.sync_copy(ones_v, acc_sh.at[dst_v.at[j]], add=True)
        return carry

    lax.fori_loop(0, NCHUNK, body, 0)
    plsc.subcore_barrier()
    orows = NPAD // NS
    pltpu.sync_copy(acc_sh.at[pl.ds(s * orows, orows)],
                    out_hbm.at[c].at[pl.ds(s * orows, orows)])


@functools.cache
def _sc_kernels():
    mesh = plsc.VectorSubcoreMesh(core_axis_name="c", subcore_axis_name="s")
    sc_aggregate = functools.partial(
        pl.kernel,
        mesh=mesh,
        out_type=jax.ShapeDtypeStruct((NC, NPAD, D), jnp.float32),
        scratch_types=[
            pltpu.VMEM((CPS, CK), jnp.int32),
            pltpu.VMEM((CPS, CK), jnp.int32),
            pltpu.VMEM((CK, D), jnp.float32),
            pltpu.VMEM((CK, D), jnp.float32),
            pltpu.VMEM_SHARED((NPAD, D), jnp.float32),
            pltpu.SemaphoreType.DMA,
            pltpu.SemaphoreType.DMA,
        ],
    )(_sc_aggregate_body)
    sc_degree = functools.partial(
        pl.kernel,
        mesh=mesh,
        out_type=jax.ShapeDtypeStruct((NC, NPAD, D), jnp.float32),
        scratch_types=[
            pltpu.VMEM((NCHUNK, CK), jnp.int32),
            pltpu.VMEM((CK, D), jnp.float32),
            pltpu.VMEM_SHARED((NPAD, D), jnp.float32),
        ],
    )(_sc_degree_body)
    return sc_aggregate, sc_degree


ROWS = 400
GRID = N // ROWS
_HI = lax.Precision.HIGHEST


def _ln_relu(h, g, b):
    m = jnp.mean(h, axis=-1, keepdims=True)
    d = h - m
    v = jnp.mean(d * d, axis=-1, keepdims=True)
    y = d * lax.rsqrt(v + 1e-5) * g + b
    return jnp.maximum(y, 0.0)


def _tc_pre_body(x_ref, wpre_ref, bpre_ref, g_ref, b_ref, h_ref, u_ref):
    h = jnp.dot(x_ref[...], wpre_ref[...], precision=_HI) + bpre_ref[...]
    h_ref[...] = h
    u_ref[...] = _ln_relu(h, g_ref[...], b_ref[...])


def _agg_update(s0, s1, d0, d1, u, h, wl, bl, wr):
    cnt = d0[:, 0:1] + d1[:, 0:1]
    inv = 1.0 / jnp.maximum(cnt, 1.0)
    agg = (s0 + s1) * inv
    return (jnp.dot(agg, wl, precision=_HI) + bl
            + jnp.dot(u, wr, precision=_HI) + h)


def _tc_mid_body(s0_ref, s1_ref, d0_ref, d1_ref, u_ref, h_ref,
                 wl_ref, bl_ref, wr_ref, g_ref, b_ref, ho_ref, uo_ref):
    hn = _agg_update(s0_ref[...], s1_ref[...], d0_ref[...], d1_ref[...],
                     u_ref[...], h_ref[...], wl_ref[...], bl_ref[...],
                     wr_ref[...])
    ho_ref[...] = hn
    uo_ref[...] = _ln_relu(hn, g_ref[...], b_ref[...])


def _tc_final_body(s0_ref, s1_ref, d0_ref, d1_ref, u_ref, h_ref,
                   wl_ref, bl_ref, wr_ref, gf_ref, bf_ref,
                   wpost_ref, bpost_ref, out_ref):
    hn = _agg_update(s0_ref[...], s1_ref[...], d0_ref[...], d1_ref[...],
                     u_ref[...], h_ref[...], wl_ref[...], bl_ref[...],
                     wr_ref[...])
    t = _ln_relu(hn, gf_ref[...], bf_ref[...])
    out_ref[...] = jnp.dot(t, wpost_ref[...], precision=_HI) + bpost_ref[...]


def _row_spec():
    return pl.BlockSpec((ROWS, D), lambda i: (i, 0))


def _deg_spec():
    return pl.BlockSpec((ROWS, 16), lambda i: (i, 0))


def _w_spec():
    return pl.BlockSpec((D, D), lambda i: (0, 0))


def _b_spec():
    return pl.BlockSpec((1, D), lambda i: (0, 0))


_tc_pre = pl.pallas_call(
    _tc_pre_body,
    grid=(GRID,),
    in_specs=[_row_spec(), _w_spec(), _b_spec(), _b_spec(), _b_spec()],
    out_specs=[_row_spec(), _row_spec()],
    out_shape=[jax.ShapeDtypeStruct((N, D), jnp.float32),
               jax.ShapeDtypeStruct((N, D), jnp.float32)],
)

_tc_mid = pl.pallas_call(
    _tc_mid_body,
    grid=(GRID,),
    in_specs=[_row_spec(), _row_spec(), _deg_spec(), _deg_spec(),
              _row_spec(), _row_spec(), _w_spec(), _b_spec(), _w_spec(),
              _b_spec(), _b_spec()],
    out_specs=[_row_spec(), _row_spec()],
    out_shape=[jax.ShapeDtypeStruct((N, D), jnp.float32),
               jax.ShapeDtypeStruct((N, D), jnp.float32)],
)

_tc_final = pl.pallas_call(
    _tc_final_body,
    grid=(GRID,),
    in_specs=[_row_spec(), _row_spec(), _deg_spec(), _deg_spec(),
              _row_spec(), _row_spec(), _w_spec(), _b_spec(), _w_spec(),
              _b_spec(), _b_spec(), _w_spec(), _b_spec()],
    out_specs=_row_spec(),
    out_shape=jax.ShapeDtypeStruct((N, D), jnp.float32),
)


def kernel(x, edge_index, W_pre, b_pre, ln_g, ln_b, Wl, bl, Wr,
           ln_gf, ln_bf, W_post, b_post):
    src = edge_index[0]
    dst = edge_index[1]
    pad = EPAD - E
    src_p = jnp.concatenate(
        [src, jnp.zeros((pad,), jnp.int32)]).reshape(NW, NCHUNK, CK)
    dummy_dst = DUMMY + (jnp.arange(pad, dtype=jnp.int32) % (NPAD - N))
    dst_p = jnp.concatenate([dst, dummy_dst]).reshape(NW, NCHUNK, CK)
    zeros_big = jnp.zeros((NPAD, D), jnp.float32)
    ones_deg = jnp.ones((CK, D), jnp.float32)

    sc_aggregate, sc_degree = _sc_kernels()
    deg = sc_degree(dst_p, zeros_big, ones_deg)
    d0 = deg[0, :, :16]
    d1 = deg[1, :, :16]

    h, u = _tc_pre(x, W_pre.T, b_pre[None], ln_g[0][None], ln_b[0][None])
    out = None
    for l in range(L):
        S = sc_aggregate(u, src_p, dst_p, zeros_big)
        if l < L - 1:
            h, u = _tc_mid(S[0], S[1], d0, d1, u, h,
                           Wl[l].T, bl[l][None], Wr[l].T,
                           ln_g[l + 1][None], ln_b[l + 1][None])
        else:
            out = _tc_final(S[0], S[1], d0, d1, u, h,
                            Wl[l].T, bl[l][None], Wr[l].T,
                            ln_gf[None], ln_bf[None],
                            W_post.T, b_post[None])
    return out


# speedup vs baseline: 8.4516x; 2.7244x over previous
import functools

import jax
import jax.numpy as jnp
from jax import lax
from jax.experimental import pallas as pl
from jax.experimental.pallas import tpu as pltpu
from jax.experimental.pallas import tpu_sc as plsc

N = 10000
E = 320000
D = 128
L = 3

NC = 2
NS = 16
NW = NC * NS
CK = 128
NCHUNK = 80
EPW = NCHUNK * CK
EPAD = NW * EPW
NPAD = 10112
DUMMY = N
NSTAGE = 2
CPS = NCHUNK // NSTAGE

def _sc_aggregate_body(u_hbm, src_hbm, dst_hbm, zeros_hbm, out_hbm,
                       src_v, dst_v, rows_a, rows_b, acc_sh, sema, semb):
    c = lax.axis_index("c")
    s = lax.axis_index("s")
    wid = s * NC + c
    zrows = NPAD // NS
    pltpu.sync_copy(zeros_hbm.at[pl.ds(s * zrows, zrows)],
                    acc_sh.at[pl.ds(s * zrows, zrows)])
    plsc.subcore_barrier()

    for st in range(NSTAGE):
        pltpu.sync_copy(src_hbm.at[wid].at[pl.ds(st * CPS, CPS)], src_v)
        pltpu.sync_copy(dst_hbm.at[wid].at[pl.ds(st * CPS, CPS)], dst_v)
        pltpu.async_copy(u_hbm.at[src_v.at[0]], rows_a, sema)

        def body(i, carry):
            j0 = 2 * i
            pltpu.async_copy(u_hbm.at[src_v.at[j0 + 1]], rows_b, semb)
            pltpu.make_async_copy(u_hbm.at[src_v.at[j0]], rows_a, sema).wait()
            pltpu.sync_copy(rows_a, acc_sh.at[dst_v.at[j0]], add=True)
            pltpu.async_copy(u_hbm.at[src_v.at[j0 + 2]], rows_a, sema)
            pltpu.make_async_copy(u_hbm.at[src_v.at[j0 + 1]], rows_b,
                                  semb).wait()
            pltpu.sync_copy(rows_b, acc_sh.at[dst_v.at[j0 + 1]], add=True)
            return carry

        lax.fori_loop(0, (CPS - 2) // 2, body, 0)
        jlast = CPS - 2
        pltpu.async_copy(u_hbm.at[src_v.at[jlast + 1]], rows_b, semb)
        pltpu.make_async_copy(u_hbm.at[src_v.at[jlast]], rows_a, sema).wait()
        pltpu.sync_copy(rows_a, acc_sh.at[dst_v.at[jlast]], add=True)
        pltpu.make_async_copy(u_hbm.at[src_v.at[jlast + 1]], rows_b,
                              semb).wait()
        pltpu.sync_copy(rows_b, acc_sh.at[dst_v.at[jlast + 1]], add=True)
    plsc.subcore_barrier()
    orows = NPAD // NS
    pltpu.sync_copy(acc_sh.at[pl.ds(s * orows, orows)],
                    out_hbm.at[c].at[pl.ds(s * orows, orows)])


def _sc_degree_body(dst_hbm, zeros_hbm, ones_hbm, out_hbm,
                    dst_v, ones_v, acc_sh):
    c = lax.axis_index("c")
    s = lax.axis_index("s")
    wid = s * NC + c
    zrows = NPAD // NS
    pltpu.sync_copy(zeros_hbm.at[pl.ds(s * zrows, zrows)],
                    acc_sh.at[pl.ds(s * zrows, zrows)])
    pltpu.sync_copy(dst_hbm.at[wid], dst_v)
    pltpu.sync_copy(ones_hbm, ones_v)
    plsc.subcore_barrier()

    def body(j, carry):
        pltpu.sync_copy(ones_v, acc_sh.at[dst_v.at[j]], add=True)
        return carry

    lax.fori_loop(0, NCHUNK, body, 0)
    plsc.subcore_barrier()
    orows = NPAD // NS
    pltpu.sync_copy(acc_sh.at[pl.ds(s * orows, orows)],
                    out_hbm.at[c].at[pl.ds(s * orows, orows)])


@functools.cache
def _sc_kernels():
    mesh = plsc.VectorSubcoreMesh(core_axis_name="c", subcore_axis_name="s")
    sc_aggregate = functools.partial(
        pl.kernel,
        mesh=mesh,
        out_type=jax.ShapeDtypeStruct((NC, NPAD, D), jnp.float32),
        scratch_types=[
            pltpu.VMEM((CPS, CK), jnp.int32),
            pltpu.VMEM((CPS, CK), jnp.int32),
            pltpu.VMEM((CK, D), jnp.float32),
            pltpu.VMEM((CK, D), jnp.float32),
            pltpu.VMEM_SHARED((NPAD, D), jnp.float32),
            pltpu.SemaphoreType.DMA,
            pltpu.SemaphoreType.DMA,
        ],
    )(_sc_aggregate_body)
    sc_degree = functools.partial(
        pl.kernel,
        mesh=mesh,
        out_type=jax.ShapeDtypeStruct((NC, NPAD, D), jnp.float32),
        scratch_types=[
            pltpu.VMEM((NCHUNK, CK), jnp.int32),
            pltpu.VMEM((CK, D), jnp.float32),
            pltpu.VMEM_SHARED((NPAD, D), jnp.float32),
        ],
    )(_sc_degree_body)
    return sc_aggregate, sc_degree


ROWS = 400
GRID = N // ROWS
_HI = lax.Precision.HIGHEST


def _ln_relu(h, g, b):
    m = jnp.mean(h, axis=-1, keepdims=True)
    d = h - m
    v = jnp.mean(d * d, axis=-1, keepdims=True)
    y = d * lax.rsqrt(v + 1e-5) * g + b
    return jnp.maximum(y, 0.0)


def _tc_pre_body(x_ref, wpre_ref, bpre_ref, g_ref, b_ref, h_ref, u_ref):
    h = jnp.dot(x_ref[...], wpre_ref[...], precision=_HI) + bpre_ref[...]
    h_ref[...] = h
    u_ref[...] = _ln_relu(h, g_ref[...], b_ref[...])


def _agg_update(s0, s1, d0, d1, u, h, wl, bl, wr):
    cnt = d0[:, 0:1] + d1[:, 0:1]
    inv = 1.0 / jnp.maximum(cnt, 1.0)
    agg = (s0 + s1) * inv
    return (jnp.dot(agg, wl, precision=_HI) + bl
            + jnp.dot(u, wr, precision=_HI) + h)


def _tc_mid_body(s0_ref, s1_ref, d0_ref, d1_ref, u_ref, h_ref,
                 wl_ref, bl_ref, wr_ref, g_ref, b_ref, ho_ref, uo_ref):
    hn = _agg_update(s0_ref[...], s1_ref[...], d0_ref[...], d1_ref[...],
                     u_ref[...], h_ref[...], wl_ref[...], bl_ref[...],
                     wr_ref[...])
    ho_ref[...] = hn
    uo_ref[...] = _ln_relu(hn, g_ref[...], b_ref[...])


def _tc_final_body(s0_ref, s1_ref, d0_ref, d1_ref, u_ref, h_ref,
                   wl_ref, bl_ref, wr_ref, gf_ref, bf_ref,
                   wpost_ref, bpost_ref, out_ref):
    hn = _agg_update(s0_ref[...], s1_ref[...], d0_ref[...], d1_ref[...],
                     u_ref[...], h_ref[...], wl_ref[...], bl_ref[...],
                     wr_ref[...])
    t = _ln_relu(hn, gf_ref[...], bf_ref[...])
    out_ref[...] = jnp.dot(t, wpost_ref[...], precision=_HI) + bpost_ref[...]


def _row_spec():
    return pl.BlockSpec((ROWS, D), lambda i: (i, 0))


def _deg_spec():
    return pl.BlockSpec((ROWS, 16), lambda i: (i, 0))


def _w_spec():
    return pl.BlockSpec((D, D), lambda i: (0, 0))


def _b_spec():
    return pl.BlockSpec((1, D), lambda i: (0, 0))


_tc_pre = pl.pallas_call(
    _tc_pre_body,
    grid=(GRID,),
    in_specs=[_row_spec(), _w_spec(), _b_spec(), _b_spec(), _b_spec()],
    out_specs=[_row_spec(), _row_spec()],
    out_shape=[jax.ShapeDtypeStruct((N, D), jnp.float32),
               jax.ShapeDtypeStruct((N, D), jnp.float32)],
)

_tc_mid = pl.pallas_call(
    _tc_mid_body,
    grid=(GRID,),
    in_specs=[_row_spec(), _row_spec(), _deg_spec(), _deg_spec(),
              _row_spec(), _row_spec(), _w_spec(), _b_spec(), _w_spec(),
              _b_spec(), _b_spec()],
    out_specs=[_row_spec(), _row_spec()],
    out_shape=[jax.ShapeDtypeStruct((N, D), jnp.float32),
               jax.ShapeDtypeStruct((N, D), jnp.float32)],
)

_tc_final = pl.pallas_call(
    _tc_final_body,
    grid=(GRID,),
    in_specs=[_row_spec(), _row_spec(), _deg_spec(), _deg_spec(),
              _row_spec(), _row_spec(), _w_spec(), _b_spec(), _w_spec(),
              _b_spec(), _b_spec(), _w_spec(), _b_spec()],
    out_specs=_row_spec(),
    out_shape=jax.ShapeDtypeStruct((N, D), jnp.float32),
)


def kernel(x, edge_index, W_pre, b_pre, ln_g, ln_b, Wl, bl, Wr,
           ln_gf, ln_bf, W_post, b_post):
    src = edge_index[0]
    dst = edge_index[1]
    epw_real = E // NW
    padw = EPW - epw_real
    w_ids = jnp.arange(NW, dtype=jnp.int32)[:, None]
    p_ids = jnp.arange(padw, dtype=jnp.int32)[None, :]
    pad_src = (w_ids * padw + p_ids) % N
    pad_dst = DUMMY + w_ids * 3 + p_ids % 3
    src_p = jnp.concatenate(
        [src.reshape(NW, epw_real), pad_src], axis=1).reshape(NW, NCHUNK, CK)
    dst_p = jnp.concatenate(
        [dst.reshape(NW, epw_real), pad_dst], axis=1).reshape(NW, NCHUNK, CK)
    zeros_big = jnp.zeros((NPAD, D), jnp.float32)
    ones_deg = jnp.ones((CK, D), jnp.float32)

    sc_aggregate, sc_degree = _sc_kernels()
    deg = sc_degree(dst_p, zeros_big, ones_deg)
    d0 = deg[0, :, :16]
    d1 = deg[1, :, :16]

    h, u = _tc_pre(x, W_pre.T, b_pre[None], ln_g[0][None], ln_b[0][None])
    out = None
    for l in range(L):
        S = sc_aggregate(u, src_p, dst_p, zeros_big)
        if l < L - 1:
            h, u = _tc_mid(S[0], S[1], d0, d1, u, h,
                           Wl[l].T, bl[l][None], Wr[l].T,
                           ln_g[l + 1][None], ln_b[l + 1][None])
        else:
            out = _tc_final(S[0], S[1], d0, d1, u, h,
                            Wl[l].T, bl[l][None], Wr[l].T,
                            ln_gf[None], ln_bf[None],
                            W_post.T, b_post[None])
    return out


# speedup vs baseline: 8.8014x; 1.0414x over previous
import functools

import jax
import jax.numpy as jnp
from jax import lax
from jax.experimental import pallas as pl
from jax.experimental.pallas import tpu as pltpu
from jax.experimental.pallas import tpu_sc as plsc

N = 10000
E = 320000
D = 128
L = 3

NC = 2
NS = 16
NW = NC * NS
CK = 128
NCHUNK = 80
EPW = NCHUNK * CK
EPAD = NW * EPW
NPAD = 10112
DUMMY = N
NSTAGE = 2
CPS = NCHUNK // NSTAGE

def _sc_aggregate_body(u_hbm, src_hbm, dst_hbm, zeros_hbm, out_hbm,
                       src_v, dst_v, rows_a, rows_b, acc_sh, sema, semb):
    c = lax.axis_index("c")
    s = lax.axis_index("s")
    wid = s * NC + c
    zrows = NPAD // NS
    pltpu.sync_copy(zeros_hbm.at[pl.ds(s * zrows, zrows)],
                    acc_sh.at[pl.ds(s * zrows, zrows)])
    plsc.subcore_barrier()

    for st in range(NSTAGE):
        pltpu.sync_copy(src_hbm.at[wid].at[pl.ds(st * CPS, CPS)], src_v)
        pltpu.sync_copy(dst_hbm.at[wid].at[pl.ds(st * CPS, CPS)], dst_v)
        pltpu.async_copy(u_hbm.at[src_v.at[0]], rows_a, sema)

        def body(i, carry):
            j0 = 2 * i
            pltpu.async_copy(u_hbm.at[src_v.at[j0 + 1]], rows_b, semb)
            pltpu.make_async_copy(u_hbm.at[src_v.at[j0]], rows_a, sema).wait()
            pltpu.sync_copy(rows_a, acc_sh.at[dst_v.at[j0]], add=True)
            pltpu.async_copy(u_hbm.at[src_v.at[j0 + 2]], rows_a, sema)
            pltpu.make_async_copy(u_hbm.at[src_v.at[j0 + 1]], rows_b,
                                  semb).wait()
            pltpu.sync_copy(rows_b, acc_sh.at[dst_v.at[j0 + 1]], add=True)
            return carry

        lax.fori_loop(0, (CPS - 2) // 2, body, 0)
        jlast = CPS - 2
        pltpu.async_copy(u_hbm.at[src_v.at[jlast + 1]], rows_b, semb)
        pltpu.make_async_copy(u_hbm.at[src_v.at[jlast]], rows_a, sema).wait()
        pltpu.sync_copy(rows_a, acc_sh.at[dst_v.at[jlast]], add=True)
        pltpu.make_async_copy(u_hbm.at[src_v.at[jlast + 1]], rows_b,
                              semb).wait()
        pltpu.sync_copy(rows_b, acc_sh.at[dst_v.at[jlast + 1]], add=True)
    plsc.subcore_barrier()
    orows = NPAD // NS
    pltpu.sync_copy(acc_sh.at[pl.ds(s * orows, orows)],
                    out_hbm.at[c].at[pl.ds(s * orows, orows)])


def _sc_degree_body(dst_hbm, zeros_hbm, ones_hbm, out_hbm,
                    dst_v, ones_v, acc_sh):
    c = lax.axis_index("c")
    s = lax.axis_index("s")
    wid = s * NC + c
    zrows = NPAD // NS
    pltpu.sync_copy(zeros_hbm.at[pl.ds(s * zrows, zrows)],
                    acc_sh.at[pl.ds(s * zrows, zrows)])
    pltpu.sync_copy(dst_hbm.at[wid], dst_v)
    pltpu.sync_copy(ones_hbm, ones_v)
    plsc.subcore_barrier()

    def body(j, carry):
        pltpu.sync_copy(ones_v, acc_sh.at[dst_v.at[j]], add=True)
        return carry

    lax.fori_loop(0, NCHUNK, body, 0)
    plsc.subcore_barrier()
    orows = NPAD // NS
    pltpu.sync_copy(acc_sh.at[pl.ds(s * orows, orows)],
                    out_hbm.at[c].at[pl.ds(s * orows, orows)])


@functools.cache
def _sc_kernels():
    mesh = plsc.VectorSubcoreMesh(core_axis_name="c", subcore_axis_name="s")
    sc_aggregate = functools.partial(
        pl.kernel,
        mesh=mesh,
        out_type=jax.ShapeDtypeStruct((NC, NPAD, D), jnp.float32),
        scratch_types=[
            pltpu.VMEM((CPS, CK), jnp.int32),
            pltpu.VMEM((CPS, CK), jnp.int32),
            pltpu.VMEM((CK, D), jnp.float32),
            pltpu.VMEM((CK, D), jnp.float32),
            pltpu.VMEM_SHARED((NPAD, D), jnp.float32),
            pltpu.SemaphoreType.DMA,
            pltpu.SemaphoreType.DMA,
        ],
    )(_sc_aggregate_body)
    sc_degree = functools.partial(
        pl.kernel,
        mesh=mesh,
        out_type=jax.ShapeDtypeStruct((NC, NPAD, D), jnp.float32),
        scratch_types=[
            pltpu.VMEM((NCHUNK, CK), jnp.int32),
            pltpu.VMEM((CK, D), jnp.float32),
            pltpu.VMEM_SHARED((NPAD, D), jnp.float32),
        ],
    )(_sc_degree_body)
    return sc_aggregate, sc_degree


ROWS = 400
GRID = N // ROWS
_HI = lax.Precision.HIGHEST


def _ln_relu(h, g, b):
    m = jnp.mean(h, axis=-1, keepdims=True)
    d = h - m
    v = jnp.mean(d * d, axis=-1, keepdims=True)
    y = d * lax.rsqrt(v + 1e-5) * g + b
    return jnp.maximum(y, 0.0)


def _tc_pre_body(x_ref, wpre_ref, bpre_ref, g_ref, b_ref, h_ref, u_ref):
    h = jnp.dot(x_ref[...], wpre_ref[...], precision=_HI) + bpre_ref[...]
    h_ref[...] = h
    u_ref[...] = _ln_relu(h, g_ref[...], b_ref[...])


def _agg_update(s_ref, d_ref, u, h, wl, bl, wr):
    cnt = d_ref[0][:, 0:1] + d_ref[1][:, 0:1]
    inv = 1.0 / jnp.maximum(cnt, 1.0)
    agg = (s_ref[0] + s_ref[1]) * inv
    return (jnp.dot(agg, wl, precision=_HI) + bl
            + jnp.dot(u, wr, precision=_HI) + h)


def _tc_mid_body(s_ref, d_ref, u_ref, h_ref,
                 wl_ref, bl_ref, wr_ref, g_ref, b_ref, ho_ref, uo_ref):
    hn = _agg_update(s_ref, d_ref, u_ref[...], h_ref[...], wl_ref[...],
                     bl_ref[...], wr_ref[...])
    ho_ref[...] = hn
    uo_ref[...] = _ln_relu(hn, g_ref[...], b_ref[...])


def _tc_final_body(s_ref, d_ref, u_ref, h_ref,
                   wl_ref, bl_ref, wr_ref, gf_ref, bf_ref,
                   wpost_ref, bpost_ref, out_ref):
    hn = _agg_update(s_ref, d_ref, u_ref[...], h_ref[...], wl_ref[...],
                     bl_ref[...], wr_ref[...])
    t = _ln_relu(hn, gf_ref[...], bf_ref[...])
    out_ref[...] = jnp.dot(t, wpost_ref[...], precision=_HI) + bpost_ref[...]


def _row_spec():
    return pl.BlockSpec((ROWS, D), lambda i: (i, 0))


def _s_spec():
    return pl.BlockSpec((NC, ROWS, D), lambda i: (0, i, 0))


def _deg_spec():
    return pl.BlockSpec((NC, ROWS, D), lambda i: (0, i, 0))


def _w_spec():
    return pl.BlockSpec((D, D), lambda i: (0, 0))


def _b_spec():
    return pl.BlockSpec((1, D), lambda i: (0, 0))


_tc_pre = pl.pallas_call(
    _tc_pre_body,
    grid=(GRID,),
    in_specs=[_row_spec(), _w_spec(), _b_spec(), _b_spec(), _b_spec()],
    out_specs=[_row_spec(), _row_spec()],
    out_shape=[jax.ShapeDtypeStruct((N, D), jnp.float32),
               jax.ShapeDtypeStruct((N, D), jnp.float32)],
)

_tc_mid = pl.pallas_call(
    _tc_mid_body,
    grid=(GRID,),
    in_specs=[_s_spec(), _deg_spec(),
              _row_spec(), _row_spec(), _w_spec(), _b_spec(), _w_spec(),
              _b_spec(), _b_spec()],
    out_specs=[_row_spec(), _row_spec()],
    out_shape=[jax.ShapeDtypeStruct((N, D), jnp.float32),
               jax.ShapeDtypeStruct((N, D), jnp.float32)],
)

_tc_final = pl.pallas_call(
    _tc_final_body,
    grid=(GRID,),
    in_specs=[_s_spec(), _deg_spec(),
              _row_spec(), _row_spec(), _w_spec(), _b_spec(), _w_spec(),
              _b_spec(), _b_spec(), _w_spec(), _b_spec()],
    out_specs=_row_spec(),
    out_shape=jax.ShapeDtypeStruct((N, D), jnp.float32),
)


def kernel(x, edge_index, W_pre, b_pre, ln_g, ln_b, Wl, bl, Wr,
           ln_gf, ln_bf, W_post, b_post):
    src = edge_index[0]
    dst = edge_index[1]
    epw_real = E // NW
    padw = EPW - epw_real
    w_ids = jnp.arange(NW, dtype=jnp.int32)[:, None]
    p_ids = jnp.arange(padw, dtype=jnp.int32)[None, :]
    pad_src = (w_ids * padw + p_ids) % N
    pad_dst = DUMMY + w_ids * 3 + p_ids % 3
    src_p = jnp.concatenate(
        [src.reshape(NW, epw_real), pad_src], axis=1).reshape(NW, NCHUNK, CK)
    dst_p = jnp.concatenate(
        [dst.reshape(NW, epw_real), pad_dst], axis=1).reshape(NW, NCHUNK, CK)
    zeros_big = jnp.zeros((NPAD, D), jnp.float32)
    ones_deg = jnp.ones((CK, D), jnp.float32)

    sc_aggregate, sc_degree = _sc_kernels()
    deg = sc_degree(dst_p, zeros_big, ones_deg)

    h, u = _tc_pre(x, W_pre.T, b_pre[None], ln_g[0][None], ln_b[0][None])
    out = None
    for l in range(L):
        S = sc_aggregate(u, src_p, dst_p, zeros_big)
        if l < L - 1:
            h, u = _tc_mid(S, deg, u, h,
                           Wl[l].T, bl[l][None], Wr[l].T,
                           ln_g[l + 1][None], ln_b[l + 1][None])
        else:
            out = _tc_final(S, deg, u, h,
                            Wl[l].T, bl[l][None], Wr[l].T,
                            ln_gf[None], ln_bf[None],
                            W_post.T, b_post[None])
    return out


# speedup vs baseline: 9.0468x; 1.0279x over previous
import functools

import jax
import jax.numpy as jnp
from jax import lax
from jax.experimental import pallas as pl
from jax.experimental.pallas import tpu as pltpu
from jax.experimental.pallas import tpu_sc as plsc

N = 10000
E = 320000
D = 128
L = 3

NC = 2
NS = 16
NW = NC * NS
CK = 128
NCHUNK = 80
EPW = NCHUNK * CK
EPAD = NW * EPW
NPAD = 10112
DUMMY = N
NSTAGE = 2
CPS = NCHUNK // NSTAGE

def _fill_rows(ref, value):
    vals = jnp.full((16,), value, jnp.float32)

    def fr(r, carry):
        for k in range(D // 16):
            ref[r, pl.ds(16 * k, 16)] = vals
        return carry

    lax.fori_loop(0, CK, fr, 0)


def _zero_acc(rows_a, acc_sh, s):
    _fill_rows(rows_a, 0.0)
    base = s * (NPAD // NS)
    for t in range((NPAD // NS) // CK):
        pltpu.sync_copy(rows_a, acc_sh.at[pl.ds(base + CK * t, CK)])
    rem = (NPAD // NS) % CK
    if rem:
        pltpu.sync_copy(rows_a.at[pl.ds(0, rem)],
                        acc_sh.at[pl.ds(base + (NPAD // NS) - rem, rem)])


def _sc_aggregate_body(u_hbm, src_hbm, dst_hbm, out_hbm,
                       src_v, dst_v, rows_a, rows_b, acc_sh, sema, semb):
    c = lax.axis_index("c")
    s = lax.axis_index("s")
    wid = s * NC + c
    _zero_acc(rows_a, acc_sh, s)
    plsc.subcore_barrier()

    for st in range(NSTAGE):
        pltpu.sync_copy(src_hbm.at[wid].at[pl.ds(st * CPS, CPS)], src_v)
        pltpu.sync_copy(dst_hbm.at[wid].at[pl.ds(st * CPS, CPS)], dst_v)
        pltpu.async_copy(u_hbm.at[src_v.at[0]], rows_a, sema)

        def body(i, carry):
            j0 = 2 * i
            pltpu.async_copy(u_hbm.at[src_v.at[j0 + 1]], rows_b, semb)
            pltpu.make_async_copy(u_hbm.at[src_v.at[j0]], rows_a, sema).wait()
            pltpu.sync_copy(rows_a, acc_sh.at[dst_v.at[j0]], add=True)
            pltpu.async_copy(u_hbm.at[src_v.at[j0 + 2]], rows_a, sema)
            pltpu.make_async_copy(u_hbm.at[src_v.at[j0 + 1]], rows_b,
                                  semb).wait()
            pltpu.sync_copy(rows_b, acc_sh.at[dst_v.at[j0 + 1]], add=True)
            return carry

        lax.fori_loop(0, (CPS - 2) // 2, body, 0)
        jlast = CPS - 2
        pltpu.async_copy(u_hbm.at[src_v.at[jlast + 1]], rows_b, semb)
        pltpu.make_async_copy(u_hbm.at[src_v.at[jlast]], rows_a, sema).wait()
        pltpu.sync_copy(rows_a, acc_sh.at[dst_v.at[jlast]], add=True)
        pltpu.make_async_copy(u_hbm.at[src_v.at[jlast + 1]], rows_b,
                              semb).wait()
        pltpu.sync_copy(rows_b, acc_sh.at[dst_v.at[jlast + 1]], add=True)
    plsc.subcore_barrier()
    orows = NPAD // NS
    pltpu.sync_copy(acc_sh.at[pl.ds(s * orows, orows)],
                    out_hbm.at[c].at[pl.ds(s * orows, orows)])


def _sc_degree_body(dst_hbm, out_hbm, dst_v, ones_v, acc_sh):
    c = lax.axis_index("c")
    s = lax.axis_index("s")
    wid = s * NC + c
    _zero_acc(ones_v, acc_sh, s)
    _fill_rows(ones_v, 1.0)
    pltpu.sync_copy(dst_hbm.at[wid], dst_v)
    plsc.subcore_barrier()

    def body(j, carry):
        pltpu.sync_copy(ones_v, acc_sh.at[dst_v.at[j]], add=True)
        return carry

    lax.fori_loop(0, NCHUNK, body, 0)
    plsc.subcore_barrier()
    orows = NPAD // NS
    pltpu.sync_copy(acc_sh.at[pl.ds(s * orows, orows)],
                    out_hbm.at[c].at[pl.ds(s * orows, orows)])


@functools.cache
def _sc_kernels():
    mesh = plsc.VectorSubcoreMesh(core_axis_name="c", subcore_axis_name="s")
    sc_aggregate = functools.partial(
        pl.kernel,
        mesh=mesh,
        out_type=jax.ShapeDtypeStruct((NC, NPAD, D), jnp.float32),
        scratch_types=[
            pltpu.VMEM((CPS, CK), jnp.int32),
            pltpu.VMEM((CPS, CK), jnp.int32),
            pltpu.VMEM((CK, D), jnp.float32),
            pltpu.VMEM((CK, D), jnp.float32),
            pltpu.VMEM_SHARED((NPAD, D), jnp.float32),
            pltpu.SemaphoreType.DMA,
            pltpu.SemaphoreType.DMA,
        ],
    )(_sc_aggregate_body)
    sc_degree = functools.partial(
        pl.kernel,
        mesh=mesh,
        out_type=jax.ShapeDtypeStruct((NC, NPAD, D), jnp.float32),
        scratch_types=[
            pltpu.VMEM((NCHUNK, CK), jnp.int32),
            pltpu.VMEM((CK, D), jnp.float32),
            pltpu.VMEM_SHARED((NPAD, D), jnp.float32),
        ],
    )(_sc_degree_body)
    return sc_aggregate, sc_degree


ROWS = 400
GRID = N // ROWS
_HI = lax.Precision.HIGHEST


def _ln_relu(h, g, b):
    m = jnp.mean(h, axis=-1, keepdims=True)
    d = h - m
    v = jnp.mean(d * d, axis=-1, keepdims=True)
    y = d * lax.rsqrt(v + 1e-5) * g + b
    return jnp.maximum(y, 0.0)


def _tc_pre_body(x_ref, wpre_ref, bpre_ref, g_ref, b_ref, h_ref, u_ref):
    h = jnp.dot(x_ref[...], wpre_ref[...], precision=_HI) + bpre_ref[...]
    h_ref[...] = h
    u_ref[...] = _ln_relu(h, g_ref[...], b_ref[...])


def _agg_update(s_ref, d_ref, u, h, wl, bl, wr):
    cnt = d_ref[0][:, 0:1] + d_ref[1][:, 0:1]
    inv = 1.0 / jnp.maximum(cnt, 1.0)
    agg = (s_ref[0] + s_ref[1]) * inv
    return (jnp.dot(agg, wl, precision=_HI) + bl
            + jnp.dot(u, wr, precision=_HI) + h)


def _tc_mid_body(s_ref, d_ref, u_ref, h_ref,
                 wl_ref, bl_ref, wr_ref, g_ref, b_ref, ho_ref, uo_ref):
    hn = _agg_update(s_ref, d_ref, u_ref[...], h_ref[...], wl_ref[...],
                     bl_ref[...], wr_ref[...])
    ho_ref[...] = hn
    uo_ref[...] = _ln_relu(hn, g_ref[...], b_ref[...])


def _tc_final_body(s_ref, d_ref, u_ref, h_ref,
                   wl_ref, bl_ref, wr_ref, gf_ref, bf_ref,
                   wpost_ref, bpost_ref, out_ref):
    hn = _agg_update(s_ref, d_ref, u_ref[...], h_ref[...], wl_ref[...],
                     bl_ref[...], wr_ref[...])
    t = _ln_relu(hn, gf_ref[...], bf_ref[...])
    out_ref[...] = jnp.dot(t, wpost_ref[...], precision=_HI) + bpost_ref[...]


def _row_spec():
    return pl.BlockSpec((ROWS, D), lambda i: (i, 0))


def _s_spec():
    return pl.BlockSpec((NC, ROWS, D), lambda i: (0, i, 0))


def _deg_spec():
    return pl.BlockSpec((NC, ROWS, 16), lambda i: (0, i, 0))


def _w_spec():
    return pl.BlockSpec((D, D), lambda i: (0, 0))


def _b_spec():
    return pl.BlockSpec((1, D), lambda i: (0, 0))


_tc_pre = pl.pallas_call(
    _tc_pre_body,
    grid=(GRID,),
    in_specs=[_row_spec(), _w_spec(), _b_spec(), _b_spec(), _b_spec()],
    out_specs=[_row_spec(), _row_spec()],
    out_shape=[jax.ShapeDtypeStruct((N, D), jnp.float32),
               jax.ShapeDtypeStruct((N, D), jnp.float32)],
)

_tc_mid = pl.pallas_call(
    _tc_mid_body,
    grid=(GRID,),
    in_specs=[_s_spec(), _deg_spec(),
              _row_spec(), _row_spec(), _w_spec(), _b_spec(), _w_spec(),
              _b_spec(), _b_spec()],
    out_specs=[_row_spec(), _row_spec()],
    out_shape=[jax.ShapeDtypeStruct((N, D), jnp.float32),
               jax.ShapeDtypeStruct((N, D), jnp.float32)],
)

_tc_final = pl.pallas_call(
    _tc_final_body,
    grid=(GRID,),
    in_specs=[_s_spec(), _deg_spec(),
              _row_spec(), _row_spec(), _w_spec(), _b_spec(), _w_spec(),
              _b_spec(), _b_spec(), _w_spec(), _b_spec()],
    out_specs=_row_spec(),
    out_shape=jax.ShapeDtypeStruct((N, D), jnp.float32),
)


def kernel(x, edge_index, W_pre, b_pre, ln_g, ln_b, Wl, bl, Wr,
           ln_gf, ln_bf, W_post, b_post):
    src = edge_index[0]
    dst = edge_index[1]
    epw_real = E // NW
    padw = EPW - epw_real
    w_ids = jnp.arange(NW, dtype=jnp.int32)[:, None]
    p_ids = jnp.arange(padw, dtype=jnp.int32)[None, :]
    pad_src = (w_ids * padw + p_ids) % N
    pad_dst = DUMMY + w_ids * 3 + p_ids % 3
    src_p = jnp.concatenate(
        [src.reshape(NW, epw_real), pad_src], axis=1).reshape(NW, NCHUNK, CK)
    dst_p = jnp.concatenate(
        [dst.reshape(NW, epw_real), pad_dst], axis=1).reshape(NW, NCHUNK, CK)
    sc_aggregate, sc_degree = _sc_kernels()
    deg = sc_degree(dst_p)[:, :, :16]

    h, u = _tc_pre(x, W_pre.T, b_pre[None], ln_g[0][None], ln_b[0][None])
    out = None
    for l in range(L):
        S = sc_aggregate(u, src_p, dst_p)
        if l < L - 1:
            h, u = _tc_mid(S, deg, u, h,
                           Wl[l].T, bl[l][None], Wr[l].T,
                           ln_g[l + 1][None], ln_b[l + 1][None])
        else:
            out = _tc_final(S, deg, u, h,
                            Wl[l].T, bl[l][None], Wr[l].T,
                            ln_gf[None], ln_bf[None],
                            W_post.T, b_post[None])
    return out


# speedup vs baseline: 9.8452x; 1.0882x over previous
import functools

import jax
import jax.numpy as jnp
from jax import lax
from jax.experimental import pallas as pl
from jax.experimental.pallas import tpu as pltpu
from jax.experimental.pallas import tpu_sc as plsc

N = 10000
E = 320000
D = 128
L = 3

NC = 2
NS = 16
NW = NC * NS
CK = 128
NCHUNK = 80
EPW = NCHUNK * CK
EPAD = NW * EPW
NPAD = 10112
DUMMY = N
NSTAGE = 2
CPS = NCHUNK // NSTAGE

def _fill_rows(ref, value):
    vals = jnp.full((16,), value, jnp.float32)

    def fr(r, carry):
        for k in range(D // 16):
            ref[r, pl.ds(16 * k, 16)] = vals
        return carry

    lax.fori_loop(0, CK, fr, 0)


def _zero_acc(rows_a, acc_sh, s):
    _fill_rows(rows_a, 0.0)
    base = s * (NPAD // NS)
    for t in range((NPAD // NS) // CK):
        pltpu.sync_copy(rows_a, acc_sh.at[pl.ds(base + CK * t, CK)])
    rem = (NPAD // NS) % CK
    if rem:
        pltpu.sync_copy(rows_a.at[pl.ds(0, rem)],
                        acc_sh.at[pl.ds(base + (NPAD // NS) - rem, rem)])


def _sc_aggregate_body(u_hbm, src_hbm, dst_hbm, out_hbm,
                       src_v, dst_v, rows_a, rows_b, acc_sh, sema, semb):
    c = lax.axis_index("c")
    s = lax.axis_index("s")
    wid = s * NC + c
    _zero_acc(rows_a, acc_sh, s)
    plsc.subcore_barrier()

    for st in range(NSTAGE):
        pltpu.sync_copy(src_hbm.at[wid].at[pl.ds(st * CPS, CPS)], src_v)
        pltpu.sync_copy(dst_hbm.at[wid].at[pl.ds(st * CPS, CPS)], dst_v)
        pltpu.async_copy(u_hbm.at[src_v.at[0]], rows_a, sema)

        def body(i, carry):
            j0 = 2 * i
            pltpu.async_copy(u_hbm.at[src_v.at[j0 + 1]], rows_b, semb)
            pltpu.make_async_copy(u_hbm.at[src_v.at[j0]], rows_a, sema).wait()
            pltpu.sync_copy(rows_a, acc_sh.at[dst_v.at[j0]], add=True)
            pltpu.async_copy(u_hbm.at[src_v.at[j0 + 2]], rows_a, sema)
            pltpu.make_async_copy(u_hbm.at[src_v.at[j0 + 1]], rows_b,
                                  semb).wait()
            pltpu.sync_copy(rows_b, acc_sh.at[dst_v.at[j0 + 1]], add=True)
            return carry

        lax.fori_loop(0, (CPS - 2) // 2, body, 0)
        jlast = CPS - 2
        pltpu.async_copy(u_hbm.at[src_v.at[jlast + 1]], rows_b, semb)
        pltpu.make_async_copy(u_hbm.at[src_v.at[jlast]], rows_a, sema).wait()
        pltpu.sync_copy(rows_a, acc_sh.at[dst_v.at[jlast]], add=True)
        pltpu.make_async_copy(u_hbm.at[src_v.at[jlast + 1]], rows_b,
                              semb).wait()
        pltpu.sync_copy(rows_b, acc_sh.at[dst_v.at[jlast + 1]], add=True)
    plsc.subcore_barrier()
    orows = NPAD // NS
    pltpu.sync_copy(acc_sh.at[pl.ds(s * orows, orows)],
                    out_hbm.at[c].at[pl.ds(s * orows, orows)])


def _sc_degree_body(dst_hbm, out_hbm, dst_v, ones_v, acc_sh):
    c = lax.axis_index("c")
    s = lax.axis_index("s")
    wid = s * NC + c
    _zero_acc(ones_v, acc_sh, s)
    _fill_rows(ones_v, 1.0)
    pltpu.sync_copy(dst_hbm.at[wid], dst_v)
    plsc.subcore_barrier()

    def body(j, carry):
        pltpu.sync_copy(ones_v, acc_sh.at[dst_v.at[j]], add=True)
        return carry

    lax.fori_loop(0, NCHUNK, body, 0)
    plsc.subcore_barrier()
    orows = NPAD // NS
    pltpu.sync_copy(acc_sh.at[pl.ds(s * orows, orows)],
                    out_hbm.at[c].at[pl.ds(s * orows, orows)])


@functools.cache
def _sc_kernels():
    mesh = plsc.VectorSubcoreMesh(core_axis_name="c", subcore_axis_name="s")
    sc_aggregate = functools.partial(
        pl.kernel,
        mesh=mesh,
        out_type=jax.ShapeDtypeStruct((NC, NPAD, D), jnp.float32),
        scratch_types=[
            pltpu.VMEM((CPS, CK), jnp.int32),
            pltpu.VMEM((CPS, CK), jnp.int32),
            pltpu.VMEM((CK, D), jnp.float32),
            pltpu.VMEM((CK, D), jnp.float32),
            pltpu.VMEM_SHARED((NPAD, D), jnp.float32),
            pltpu.SemaphoreType.DMA,
            pltpu.SemaphoreType.DMA,
        ],
    )(_sc_aggregate_body)
    sc_degree = functools.partial(
        pl.kernel,
        mesh=mesh,
        out_type=jax.ShapeDtypeStruct((NC, NPAD, D), jnp.float32),
        scratch_types=[
            pltpu.VMEM((NCHUNK, CK), jnp.int32),
            pltpu.VMEM((CK, D), jnp.float32),
            pltpu.VMEM_SHARED((NPAD, D), jnp.float32),
        ],
    )(_sc_degree_body)
    return sc_aggregate, sc_degree


ROWS = 2000
GRID = N // ROWS
_HI = lax.Precision.HIGHEST


def _ln_relu(h, g, b):
    m = jnp.mean(h, axis=-1, keepdims=True)
    d = h - m
    v = jnp.mean(d * d, axis=-1, keepdims=True)
    y = d * lax.rsqrt(v + 1e-5) * g + b
    return jnp.maximum(y, 0.0)


def _tc_pre_body(x_ref, wpre_ref, bpre_ref, g_ref, b_ref, h_ref, u_ref):
    h = jnp.dot(x_ref[...], wpre_ref[...], precision=_HI) + bpre_ref[...]
    h_ref[...] = h
    u_ref[...] = _ln_relu(h, g_ref[...], b_ref[...])


def _agg_update(s_ref, d_ref, u, h, wl, bl, wr):
    cnt = d_ref[0][:, 0:1] + d_ref[1][:, 0:1]
    inv = 1.0 / jnp.maximum(cnt, 1.0)
    agg = (s_ref[0] + s_ref[1]) * inv
    return (jnp.dot(agg, wl, precision=_HI) + bl
            + jnp.dot(u, wr, precision=_HI) + h)


def _tc_mid_body(s_ref, d_ref, u_ref, h_ref,
                 wl_ref, bl_ref, wr_ref, g_ref, b_ref, ho_ref, uo_ref):
    hn = _agg_update(s_ref, d_ref, u_ref[...], h_ref[...], wl_ref[...],
                     bl_ref[...], wr_ref[...])
    ho_ref[...] = hn
    uo_ref[...] = _ln_relu(hn, g_ref[...], b_ref[...])


def _tc_final_body(s_ref, d_ref, u_ref, h_ref,
                   wl_ref, bl_ref, wr_ref, gf_ref, bf_ref,
                   wpost_ref, bpost_ref, out_ref):
    hn = _agg_update(s_ref, d_ref, u_ref[...], h_ref[...], wl_ref[...],
                     bl_ref[...], wr_ref[...])
    t = _ln_relu(hn, gf_ref[...], bf_ref[...])
    out_ref[...] = jnp.dot(t, wpost_ref[...], precision=_HI) + bpost_ref[...]


def _row_spec():
    return pl.BlockSpec((ROWS, D), lambda i: (i, 0))


def _s_spec():
    return pl.BlockSpec((NC, ROWS, D), lambda i: (0, i, 0))


def _deg_spec():
    return pl.BlockSpec((NC, ROWS, 16), lambda i: (0, i, 0))


def _w_spec():
    return pl.BlockSpec((D, D), lambda i: (0, 0))


def _b_spec():
    return pl.BlockSpec((1, D), lambda i: (0, 0))


_tc_pre = pl.pallas_call(
    _tc_pre_body,
    grid=(GRID,),
    in_specs=[_row_spec(), _w_spec(), _b_spec(), _b_spec(), _b_spec()],
    out_specs=[_row_spec(), _row_spec()],
    out_shape=[jax.ShapeDtypeStruct((N, D), jnp.float32),
               jax.ShapeDtypeStruct((N, D), jnp.float32)],
)

_tc_mid = pl.pallas_call(
    _tc_mid_body,
    grid=(GRID,),
    in_specs=[_s_spec(), _deg_spec(),
              _row_spec(), _row_spec(), _w_spec(), _b_spec(), _w_spec(),
              _b_spec(), _b_spec()],
    out_specs=[_row_spec(), _row_spec()],
    out_shape=[jax.ShapeDtypeStruct((N, D), jnp.float32),
               jax.ShapeDtypeStruct((N, D), jnp.float32)],
)

_tc_final = pl.pallas_call(
    _tc_final_body,
    grid=(GRID,),
    in_specs=[_s_spec(), _deg_spec(),
              _row_spec(), _row_spec(), _w_spec(), _b_spec(), _w_spec(),
              _b_spec(), _b_spec(), _w_spec(), _b_spec()],
    out_specs=_row_spec(),
    out_shape=jax.ShapeDtypeStruct((N, D), jnp.float32),
)


def kernel(x, edge_index, W_pre, b_pre, ln_g, ln_b, Wl, bl, Wr,
           ln_gf, ln_bf, W_post, b_post):
    src = edge_index[0]
    dst = edge_index[1]
    epw_real = E // NW
    padw = EPW - epw_real
    w_ids = jnp.arange(NW, dtype=jnp.int32)[:, None]
    p_ids = jnp.arange(padw, dtype=jnp.int32)[None, :]
    pad_src = (w_ids * padw + p_ids) % N
    pad_dst = DUMMY + w_ids * 3 + p_ids % 3
    src_p = jnp.concatenate(
        [src.reshape(NW, epw_real), pad_src], axis=1).reshape(NW, NCHUNK, CK)
    dst_p = jnp.concatenate(
        [dst.reshape(NW, epw_real), pad_dst], axis=1).reshape(NW, NCHUNK, CK)
    sc_aggregate, sc_degree = _sc_kernels()
    deg = sc_degree(dst_p)[:, :, :16]

    h, u = _tc_pre(x, W_pre.T, b_pre[None], ln_g[0][None], ln_b[0][None])
    out = None
    for l in range(L):
        S = sc_aggregate(u, src_p, dst_p)
        if l < L - 1:
            h, u = _tc_mid(S, deg, u, h,
                           Wl[l].T, bl[l][None], Wr[l].T,
                           ln_g[l + 1][None], ln_b[l + 1][None])
        else:
            out = _tc_final(S, deg, u, h,
                            Wl[l].T, bl[l][None], Wr[l].T,
                            ln_gf[None], ln_bf[None],
                            W_post.T, b_post[None])
    return out


# speedup vs baseline: 9.9341x; 1.0090x over previous
import functools

import jax
import jax.numpy as jnp
from jax import lax
from jax.experimental import pallas as pl
from jax.experimental.pallas import tpu as pltpu
from jax.experimental.pallas import tpu_sc as plsc

N = 10000
E = 320000
D = 128
L = 3

NC = 2
NS = 16
NW = NC * NS
CK = 128
NCHUNK = 80
EPW = NCHUNK * CK
EPAD = NW * EPW
NPAD = 10112
DUMMY = N
NSTAGE = 2
CPS = NCHUNK // NSTAGE

def _fill_rows(ref, value):
    vals = jnp.full((16,), value, jnp.float32)

    def fr(r, carry):
        for k in range(D // 16):
            ref[r, pl.ds(16 * k, 16)] = vals
        return carry

    lax.fori_loop(0, CK, fr, 0)


def _zero_acc(rows_a, acc_sh, s):
    _fill_rows(rows_a, 0.0)
    base = s * (NPAD // NS)
    for t in range((NPAD // NS) // CK):
        pltpu.sync_copy(rows_a, acc_sh.at[pl.ds(base + CK * t, CK)])
    rem = (NPAD // NS) % CK
    if rem:
        pltpu.sync_copy(rows_a.at[pl.ds(0, rem)],
                        acc_sh.at[pl.ds(base + (NPAD // NS) - rem, rem)])


def _agg_loop(u_hbm, src_hbm, dst_hbm, src_v, dst_v, rows_a, rows_b,
              acc_sh, sema, semb, wid):
    for st in range(NSTAGE):
        pltpu.sync_copy(src_hbm.at[wid].at[pl.ds(st * CPS, CPS)], src_v)
        pltpu.sync_copy(dst_hbm.at[wid].at[pl.ds(st * CPS, CPS)], dst_v)
        pltpu.async_copy(u_hbm.at[src_v.at[0]], rows_a, sema)

        def body(i, carry):
            j0 = 2 * i
            pltpu.async_copy(u_hbm.at[src_v.at[j0 + 1]], rows_b, semb)
            pltpu.make_async_copy(u_hbm.at[src_v.at[j0]], rows_a, sema).wait()
            pltpu.sync_copy(rows_a, acc_sh.at[dst_v.at[j0]], add=True)
            pltpu.async_copy(u_hbm.at[src_v.at[j0 + 2]], rows_a, sema)
            pltpu.make_async_copy(u_hbm.at[src_v.at[j0 + 1]], rows_b,
                                  semb).wait()
            pltpu.sync_copy(rows_b, acc_sh.at[dst_v.at[j0 + 1]], add=True)
            return carry

        lax.fori_loop(0, (CPS - 2) // 2, body, 0)
        jlast = CPS - 2
        pltpu.async_copy(u_hbm.at[src_v.at[jlast + 1]], rows_b, semb)
        pltpu.make_async_copy(u_hbm.at[src_v.at[jlast]], rows_a, sema).wait()
        pltpu.sync_copy(rows_a, acc_sh.at[dst_v.at[jlast]], add=True)
        pltpu.make_async_copy(u_hbm.at[src_v.at[jlast + 1]], rows_b,
                              semb).wait()
        pltpu.sync_copy(rows_b, acc_sh.at[dst_v.at[jlast + 1]], add=True)


def _sc_aggregate_body(u_hbm, src_hbm, dst_hbm, out_hbm,
                       src_v, dst_v, rows_a, rows_b, acc_sh, sema, semb):
    c = lax.axis_index("c")
    s = lax.axis_index("s")
    wid = s * NC + c
    _zero_acc(rows_a, acc_sh, s)
    plsc.subcore_barrier()
    _agg_loop(u_hbm, src_hbm, dst_hbm, src_v, dst_v, rows_a, rows_b,
              acc_sh, sema, semb, wid)
    plsc.subcore_barrier()
    orows = NPAD // NS
    pltpu.sync_copy(acc_sh.at[pl.ds(s * orows, orows)],
                    out_hbm.at[c].at[pl.ds(s * orows, orows)])


def _sc_agg_deg_body(u_hbm, src_hbm, dst_hbm, s_out, deg_out,
                     src_v, dst_v, rows_a, rows_b, acc_sh, sema, semb):
    c = lax.axis_index("c")
    s = lax.axis_index("s")
    wid = s * NC + c
    orows = NPAD // NS
    _zero_acc(rows_a, acc_sh, s)
    _fill_rows(rows_a, 1.0)
    plsc.subcore_barrier()
    for st in range(NSTAGE):
        pltpu.sync_copy(dst_hbm.at[wid].at[pl.ds(st * CPS, CPS)], dst_v)

        def dbody(j, carry):
            pltpu.sync_copy(rows_a, acc_sh.at[dst_v.at[j]], add=True)
            return carry

        lax.fori_loop(0, CPS, dbody, 0)
    plsc.subcore_barrier()
    pltpu.sync_copy(acc_sh.at[pl.ds(s * orows, orows)],
                    deg_out.at[c].at[pl.ds(s * orows, orows)])
    _zero_acc(rows_a, acc_sh, s)
    plsc.subcore_barrier()
    _agg_loop(u_hbm, src_hbm, dst_hbm, src_v, dst_v, rows_a, rows_b,
              acc_sh, sema, semb, wid)
    plsc.subcore_barrier()
    pltpu.sync_copy(acc_sh.at[pl.ds(s * orows, orows)],
                    s_out.at[c].at[pl.ds(s * orows, orows)])


@functools.cache
def _sc_kernels():
    mesh = plsc.VectorSubcoreMesh(core_axis_name="c", subcore_axis_name="s")
    sc_aggregate = functools.partial(
        pl.kernel,
        mesh=mesh,
        out_type=jax.ShapeDtypeStruct((NC, NPAD, D), jnp.float32),
        scratch_types=[
            pltpu.VMEM((CPS, CK), jnp.int32),
            pltpu.VMEM((CPS, CK), jnp.int32),
            pltpu.VMEM((CK, D), jnp.float32),
            pltpu.VMEM((CK, D), jnp.float32),
            pltpu.VMEM_SHARED((NPAD, D), jnp.float32),
            pltpu.SemaphoreType.DMA,
            pltpu.SemaphoreType.DMA,
        ],
    )(_sc_aggregate_body)
    sc_agg_deg = functools.partial(
        pl.kernel,
        mesh=mesh,
        out_type=[jax.ShapeDtypeStruct((NC, NPAD, D), jnp.float32),
                  jax.ShapeDtypeStruct((NC, NPAD, D), jnp.float32)],
        scratch_types=[
            pltpu.VMEM((CPS, CK), jnp.int32),
            pltpu.VMEM((CPS, CK), jnp.int32),
            pltpu.VMEM((CK, D), jnp.float32),
            pltpu.VMEM((CK, D), jnp.float32),
            pltpu.VMEM_SHARED((NPAD, D), jnp.float32),
            pltpu.SemaphoreType.DMA,
            pltpu.SemaphoreType.DMA,
        ],
    )(_sc_agg_deg_body)
    return sc_aggregate, sc_agg_deg


ROWS = 2000
GRID = N // ROWS
_HI = lax.Precision.HIGHEST


def _ln_relu(h, g, b):
    m = jnp.mean(h, axis=-1, keepdims=True)
    d = h - m
    v = jnp.mean(d * d, axis=-1, keepdims=True)
    y = d * lax.rsqrt(v + 1e-5) * g + b
    return jnp.maximum(y, 0.0)


def _tc_pre_body(x_ref, wpre_ref, bpre_ref, g_ref, b_ref, h_ref, u_ref):
    h = jnp.dot(x_ref[...], wpre_ref[...], precision=_HI) + bpre_ref[...]
    h_ref[...] = h
    u_ref[...] = _ln_relu(h, g_ref[...], b_ref[...])


def _agg_update(s_ref, d_ref, u, h, wl, bl, wr):
    cnt = d_ref[0][:, 0:1] + d_ref[1][:, 0:1]
    inv = 1.0 / jnp.maximum(cnt, 1.0)
    agg = (s_ref[0] + s_ref[1]) * inv
    return (jnp.dot(agg, wl, precision=_HI) + bl
            + jnp.dot(u, wr, precision=_HI) + h)


def _tc_mid_body(s_ref, d_ref, u_ref, h_ref,
                 wl_ref, bl_ref, wr_ref, g_ref, b_ref, ho_ref, uo_ref):
    hn = _agg_update(s_ref, d_ref, u_ref[...], h_ref[...], wl_ref[...],
                     bl_ref[...], wr_ref[...])
    ho_ref[...] = hn
    uo_ref[...] = _ln_relu(hn, g_ref[...], b_ref[...])


def _tc_final_body(s_ref, d_ref, u_ref, h_ref,
                   wl_ref, bl_ref, wr_ref, gf_ref, bf_ref,
                   wpost_ref, bpost_ref, out_ref):
    hn = _agg_update(s_ref, d_ref, u_ref[...], h_ref[...], wl_ref[...],
                     bl_ref[...], wr_ref[...])
    t = _ln_relu(hn, gf_ref[...], bf_ref[...])
    out_ref[...] = jnp.dot(t, wpost_ref[...], precision=_HI) + bpost_ref[...]


def _row_spec():
    return pl.BlockSpec((ROWS, D), lambda i: (i, 0))


def _s_spec():
    return pl.BlockSpec((NC, ROWS, D), lambda i: (0, i, 0))


def _deg_spec():
    return pl.BlockSpec((NC, ROWS, 16), lambda i: (0, i, 0))


def _w_spec():
    return pl.BlockSpec((D, D), lambda i: (0, 0))


def _b_spec():
    return pl.BlockSpec((1, D), lambda i: (0, 0))


_tc_pre = pl.pallas_call(
    _tc_pre_body,
    grid=(GRID,),
    in_specs=[_row_spec(), _w_spec(), _b_spec(), _b_spec(), _b_spec()],
    out_specs=[_row_spec(), _row_spec()],
    out_shape=[jax.ShapeDtypeStruct((N, D), jnp.float32),
               jax.ShapeDtypeStruct((N, D), jnp.float32)],
)

_tc_mid = pl.pallas_call(
    _tc_mid_body,
    grid=(GRID,),
    in_specs=[_s_spec(), _deg_spec(),
              _row_spec(), _row_spec(), _w_spec(), _b_spec(), _w_spec(),
              _b_spec(), _b_spec()],
    out_specs=[_row_spec(), _row_spec()],
    out_shape=[jax.ShapeDtypeStruct((N, D), jnp.float32),
               jax.ShapeDtypeStruct((N, D), jnp.float32)],
)

_tc_final = pl.pallas_call(
    _tc_final_body,
    grid=(GRID,),
    in_specs=[_s_spec(), _deg_spec(),
              _row_spec(), _row_spec(), _w_spec(), _b_spec(), _w_spec(),
              _b_spec(), _b_spec(), _w_spec(), _b_spec()],
    out_specs=_row_spec(),
    out_shape=jax.ShapeDtypeStruct((N, D), jnp.float32),
)


def kernel(x, edge_index, W_pre, b_pre, ln_g, ln_b, Wl, bl, Wr,
           ln_gf, ln_bf, W_post, b_post):
    src = edge_index[0]
    dst = edge_index[1]
    epw_real = E // NW
    padw = EPW - epw_real
    w_ids = jnp.arange(NW, dtype=jnp.int32)[:, None]
    p_ids = jnp.arange(padw, dtype=jnp.int32)[None, :]
    pad_src = (w_ids * padw + p_ids) % N
    pad_dst = DUMMY + w_ids * 3 + p_ids % 3
    src_p = jnp.concatenate(
        [src.reshape(NW, epw_real), pad_src], axis=1).reshape(NW, NCHUNK, CK)
    dst_p = jnp.concatenate(
        [dst.reshape(NW, epw_real), pad_dst], axis=1).reshape(NW, NCHUNK, CK)
    sc_aggregate, sc_agg_deg = _sc_kernels()

    h, u = _tc_pre(x, W_pre.T, b_pre[None], ln_g[0][None], ln_b[0][None])
    out = None
    deg = None
    for l in range(L):
        if l == 0:
            S, degf = sc_agg_deg(u, src_p, dst_p)
            deg = degf[:, :, :16]
        else:
            S = sc_aggregate(u, src_p, dst_p)
        if l < L - 1:
            h, u = _tc_mid(S, deg, u, h,
                           Wl[l].T, bl[l][None], Wr[l].T,
                           ln_g[l + 1][None], ln_b[l + 1][None])
        else:
            out = _tc_final(S, deg, u, h,
                            Wl[l].T, bl[l][None], Wr[l].T,
                            ln_gf[None], ln_bf[None],
                            W_post.T, b_post[None])
    return out
